# baseline (speedup 1.0000x reference)
import jax
import jax.numpy as jnp
from jax.experimental import pallas as pl
from jax.experimental.pallas import tpu as pltpu

_B = 4096
_D = 1024
_N = 2 * _B
_TEMP = 0.5
_INV = 1.0 / (_D * _TEMP)
_LOG2E = 1.4426950408889634
_ROWSCALE = 2.0 * _INV * _LOG2E

_BL = 512
_BQ = 1024
_NQ = _B // _BQ
_NR = _N // _BL


def _sq_row(z32):
    zsq = z32 * z32
    ones = jnp.ones((8, _D), dtype=jnp.float32)
    s = jax.lax.dot_general(
        ones, zsq, (((1,), (1,)), ((), ())),
        preferred_element_type=jnp.float32)
    return s[0:1, :]


def _prep_kernel(zi_ref, zj_ref, zb_ref, sqs_ref):
    zi = zi_ref[...]
    zj = zj_ref[...]
    zb_ref[0] = zi.astype(jnp.bfloat16)
    zb_ref[1] = zj.astype(jnp.bfloat16)
    sqs_ref[0] = _sq_row(zi) * jnp.float32(_INV * _LOG2E)
    sqs_ref[1] = _sq_row(zj) * jnp.float32(_INV * _LOG2E)


def _loss_kernel(zb_ref, sqs_ref, out_ref, rowacc_ref, colacc_ref, zrs_ref,
                 wrep_ref):
    r = pl.program_id(0)
    h = r // (_NR // 2)
    off = pl.multiple_of(jax.lax.rem(r, _NR // 2) * _BL, _BL)
    zr = zb_ref[h, pl.ds(off, _BL), :]
    zr32 = zr.astype(jnp.float32)
    s2 = (jnp.sum(zr32 * zr32, axis=1, keepdims=True)
          * jnp.float32(_INV * _LOG2E))
    wrep_ref[...] = jnp.broadcast_to(jnp.exp2(s2), (_BL, 128))
    zrs_ref[...] = zr * jnp.bfloat16(_ROWSCALE)

    @pl.when(r == 0)
    def _():
        colacc_ref[...] = jnp.zeros_like(colacc_ref)

    rowacc_ref[...] = jnp.zeros_like(rowacc_ref)

    for c in range(_NR):
        @pl.when(c >= r)
        def _(c=c):
            hc, oc = c // (_NR // 2), (c % (_NR // 2)) * _BL
            zc = zb_ref[hc, oc:oc + _BL, :]
            g = jax.lax.dot_general(
                zrs_ref[...], zc, (((1,), (1,)), ((), ())),
                preferred_element_type=jnp.float32)
            sqs_c = sqs_ref[hc, 0:1, oc:oc + _BL]
            ex = jnp.exp2(sqs_c - g)
            rsum = jnp.sum(ex, axis=1, keepdims=True)
            rowacc_ref[...] += jnp.transpose(rsum, (1, 0))

            @pl.when(c > r)
            def _():
                w512 = pltpu.repeat(wrep_ref[...], 4, axis=1)
                csum = jnp.sum(ex * w512, axis=0, keepdims=True)
                colacc_ref[0:1, c * _BL:(c + 1) * _BL] += csum

    tcols = colacc_ref[0:1, pl.ds(pl.multiple_of(r * _BL, _BL), _BL)]
    w_lane = jnp.exp2(sqs_ref[h, 0:1, pl.ds(off, _BL)])
    row_sum = w_lane * rowacc_ref[...] + tcols - 1.0
    lse = jnp.log(row_sum)

    zp32 = zb_ref[1 - h, pl.ds(off, _BL), :].astype(jnp.float32)
    diff = zr32 - zp32
    pos = jnp.sum(diff * diff, axis=1, keepdims=True) * _INV
    posT = jnp.transpose(pos, (1, 0))

    total = jnp.sum(lse - posT, axis=1, keepdims=True)
    out_ref[...] = jnp.broadcast_to(total[None], (1, 1, 128))


def kernel(z_i, z_j):
    zb, sqs = pl.pallas_call(
        _prep_kernel,
        grid=(_NQ,),
        in_specs=[
            pl.BlockSpec((_BQ, _D), lambda q: (q, 0)),
            pl.BlockSpec((_BQ, _D), lambda q: (q, 0)),
        ],
        out_specs=[
            pl.BlockSpec((2, _BQ, _D), lambda q: (0, q, 0)),
            pl.BlockSpec((2, 1, _BQ), lambda q: (0, 0, q)),
        ],
        out_shape=[
            jax.ShapeDtypeStruct((2, _B, _D), jnp.bfloat16),
            jax.ShapeDtypeStruct((2, 1, _B), jnp.float32),
        ],
        compiler_params=pltpu.CompilerParams(
            dimension_semantics=("arbitrary",),
        ),
    )(z_i, z_j)

    partials = pl.pallas_call(
        _loss_kernel,
        grid=(_NR,),
        in_specs=[
            pl.BlockSpec((2, _B, _D), lambda r: (0, 0, 0)),
            pl.BlockSpec((2, 1, _B), lambda r: (0, 0, 0)),
        ],
        out_specs=pl.BlockSpec((1, 1, 128), lambda r: (r, 0, 0)),
        out_shape=jax.ShapeDtypeStruct((_NR, 1, 128), jnp.float32),
        scratch_shapes=[
            pltpu.VMEM((1, _BL), jnp.float32),
            pltpu.VMEM((1, _N), jnp.float32),
            pltpu.VMEM((_BL, _D), jnp.bfloat16),
            pltpu.VMEM((_BL, 128), jnp.float32),
        ],
        compiler_params=pltpu.CompilerParams(
            dimension_semantics=("arbitrary",),
            vmem_limit_bytes=50331648,
        ),
    )(zb, sqs)

    return jnp.sum(partials[:, 0, 0]) * (1.0 / _N)


# speedup vs baseline: 1.1313x; 1.1313x over previous
import jax
import jax.numpy as jnp
from jax.experimental import pallas as pl
from jax.experimental.pallas import tpu as pltpu

_B = 4096
_D = 1024
_N = 2 * _B
_TEMP = 0.5
_INV = 1.0 / (_D * _TEMP)
_LOG2E = 1.4426950408889634
_ROWSCALE = 2.0 * _INV * _LOG2E

_BL = 512
_BQ = 1024
_NQ = _B // _BQ
_NR = _N // _BL


def _ones_dot(x32):
    ones = jnp.ones((8, _D), dtype=jnp.float32)
    s = jax.lax.dot_general(
        ones, x32, (((1,), (1,)), ((), ())),
        preferred_element_type=jnp.float32)
    return s[0:1, :]


def _prep_kernel(zi_ref, zj_ref, zb_ref, sqs_ref, posv_ref):
    zi = zi_ref[...]
    zj = zj_ref[...]
    zb_ref[0] = zi.astype(jnp.bfloat16)
    zb_ref[1] = zj.astype(jnp.bfloat16)
    sqs_ref[0] = _ones_dot(zi * zi) * jnp.float32(_INV * _LOG2E)
    sqs_ref[1] = _ones_dot(zj * zj) * jnp.float32(_INV * _LOG2E)
    dij = zi - zj
    posv_ref[0] = _ones_dot(dij * dij) * jnp.float32(_INV)


def _loss_kernel(zb_ref, sqs_ref, posv_ref, out_ref, acc_ref, colacc_ref,
                 zrs_ref, wrep_ref):
    r = pl.program_id(0)
    h = r // (_NR // 2)
    off = pl.multiple_of(jax.lax.rem(r, _NR // 2) * _BL, _BL)
    zrs_ref[...] = zb_ref[h, pl.ds(off, _BL), :] * jnp.bfloat16(_ROWSCALE)
    w_lane = jnp.exp2(sqs_ref[h, 0:1, pl.ds(off, _BL)])
    wrep_ref[...] = jnp.broadcast_to(
        jnp.transpose(w_lane, (1, 0)), (_BL, 128))

    @pl.when(r == 0)
    def _():
        colacc_ref[...] = jnp.zeros_like(colacc_ref)

    acc_ref[...] = jnp.zeros_like(acc_ref)

    for c in range(_NR):
        @pl.when(c >= r)
        def _(c=c):
            hc, oc = c // (_NR // 2), (c % (_NR // 2)) * _BL
            zc = zb_ref[hc, oc:oc + _BL, :]
            g = jax.lax.dot_general(
                zrs_ref[...], zc, (((1,), (1,)), ((), ())),
                preferred_element_type=jnp.float32)
            sqs_c = sqs_ref[hc, 0:1, oc:oc + _BL]
            ex = jnp.exp2(sqs_c - g)
            acc_ref[...] += ((ex[:, 0:128] + ex[:, 128:256])
                             + (ex[:, 256:384] + ex[:, 384:512]))

            @pl.when(c > r)
            def _():
                w512 = pltpu.repeat(wrep_ref[...], 4, axis=1)
                csum = jnp.sum(ex * w512, axis=0, keepdims=True)
                colacc_ref[0:1, c * _BL:(c + 1) * _BL] += csum

    tcols = colacc_ref[0:1, pl.ds(pl.multiple_of(r * _BL, _BL), _BL)]
    tcolsT = jnp.transpose(tcols, (1, 0))

    p_local = jnp.sum(acc_ref[...], axis=1, keepdims=True)
    w_col = wrep_ref[:, 0:1]
    row_sum = w_col * p_local + tcolsT - 1.0
    lse = jnp.log(row_sum)

    total_lse = jnp.sum(lse, axis=0, keepdims=True)
    pos_lane = posv_ref[0, 0:1, pl.ds(off, _BL)]
    total = total_lse - jnp.sum(pos_lane, axis=1, keepdims=True)
    out_ref[...] = jnp.broadcast_to(total[None], (1, 1, 128))


def kernel(z_i, z_j):
    zb, sqs, posv = pl.pallas_call(
        _prep_kernel,
        grid=(_NQ,),
        in_specs=[
            pl.BlockSpec((_BQ, _D), lambda q: (q, 0)),
            pl.BlockSpec((_BQ, _D), lambda q: (q, 0)),
        ],
        out_specs=[
            pl.BlockSpec((2, _BQ, _D), lambda q: (0, q, 0)),
            pl.BlockSpec((2, 1, _BQ), lambda q: (0, 0, q)),
            pl.BlockSpec((1, 1, _BQ), lambda q: (0, 0, q)),
        ],
        out_shape=[
            jax.ShapeDtypeStruct((2, _B, _D), jnp.bfloat16),
            jax.ShapeDtypeStruct((2, 1, _B), jnp.float32),
            jax.ShapeDtypeStruct((1, 1, _B), jnp.float32),
        ],
        compiler_params=pltpu.CompilerParams(
            dimension_semantics=("arbitrary",),
        ),
    )(z_i, z_j)

    partials = pl.pallas_call(
        _loss_kernel,
        grid=(_NR,),
        in_specs=[
            pl.BlockSpec((2, _B, _D), lambda r: (0, 0, 0)),
            pl.BlockSpec((2, 1, _B), lambda r: (0, 0, 0)),
            pl.BlockSpec((1, 1, _B), lambda r: (0, 0, 0)),
        ],
        out_specs=pl.BlockSpec((1, 1, 128), lambda r: (r, 0, 0)),
        out_shape=jax.ShapeDtypeStruct((_NR, 1, 128), jnp.float32),
        scratch_shapes=[
            pltpu.VMEM((_BL, 128), jnp.float32),
            pltpu.VMEM((1, _N), jnp.float32),
            pltpu.VMEM((_BL, _D), jnp.bfloat16),
            pltpu.VMEM((_BL, 128), jnp.float32),
        ],
        compiler_params=pltpu.CompilerParams(
            dimension_semantics=("arbitrary",),
            vmem_limit_bytes=50331648,
        ),
    )(zb, sqs, posv)

    return jnp.sum(partials[:, 0, 0]) * (1.0 / _N)
